# Initial kernel scaffold; baseline (speedup 1.0000x reference)
import jax
import jax.numpy as jnp
from jax.experimental import pallas as pl


def kernel(predictions, anchors):
    raise NotImplementedError("write your pallas kernel here")


# speedup vs baseline: 1.8146x; 1.8146x over previous
import jax
import jax.numpy as jnp
from jax.experimental import pallas as pl

_NMS_IOU = 0.1
_MAX_PER_CLASS = 50
_MAX_TOTAL = 50
_K_PRE = 1000
_NEG_INF = float("-inf")


def _score_mask_kernel(logit_ref, out_ref):
    p = jax.nn.sigmoid(logit_ref[...])
    out_ref[...] = jnp.where(p >= 0.5, p, _NEG_INF)


def _nms_kernel(s_ref, px_ref, py_ref, pw_ref, ph_ref,
                ax_ref, ay_ref, aw_ref, ah_ref,
                os_ref, ox1_ref, oy1_ref, ox2_ref, oy2_ref):
    s = s_ref[...]
    cx = px_ref[...] * aw_ref[...] + ax_ref[...]
    cy = py_ref[...] * ah_ref[...] + ay_ref[...]
    w = jnp.exp(pw_ref[...]) * aw_ref[...]
    h = jnp.exp(ph_ref[...]) * ah_ref[...]
    x1 = cx - w / 2.0
    y1 = cy - h / 2.0
    x2 = cx + w / 2.0
    y2 = cy + h / 2.0
    area = jnp.maximum(x2 - x1, 0.0) * jnp.maximum(y2 - y1, 0.0)

    iota = jax.lax.broadcasted_iota(jnp.int32, s.shape, 1)
    big = jnp.int32(1 << 30)
    alive = jnp.ones(s.shape, jnp.bool_)

    for t in range(_MAX_PER_CLASS):
        idxv = jnp.min(jnp.where(alive, iota, big), axis=1, keepdims=True)
        oh = iota == idxv
        sel_s = jnp.max(jnp.where(oh, s, _NEG_INF), axis=1, keepdims=True)
        valid = sel_s > _NEG_INF
        sel_x1 = jnp.sum(jnp.where(oh, x1, 0.0), axis=1, keepdims=True)
        sel_y1 = jnp.sum(jnp.where(oh, y1, 0.0), axis=1, keepdims=True)
        sel_x2 = jnp.sum(jnp.where(oh, x2, 0.0), axis=1, keepdims=True)
        sel_y2 = jnp.sum(jnp.where(oh, y2, 0.0), axis=1, keepdims=True)
        sel_a = jnp.sum(jnp.where(oh, area, 0.0), axis=1, keepdims=True)

        ix1 = jnp.maximum(sel_x1, x1)
        iy1 = jnp.maximum(sel_y1, y1)
        ix2 = jnp.minimum(sel_x2, x2)
        iy2 = jnp.minimum(sel_y2, y2)
        inter = jnp.maximum(ix2 - ix1, 0.0) * jnp.maximum(iy2 - iy1, 0.0)
        iou = inter / (sel_a + area - inter + 1e-8)
        alive = alive & ~((iou > _NMS_IOU) & valid) & ~oh

        os_ref[:, t:t + 1] = jnp.where(valid, sel_s, _NEG_INF)
        ox1_ref[:, t:t + 1] = jnp.where(valid, sel_x1, 0.0)
        oy1_ref[:, t:t + 1] = jnp.where(valid, sel_y1, 0.0)
        ox2_ref[:, t:t + 1] = jnp.where(valid, sel_x2, 0.0)
        oy2_ref[:, t:t + 1] = jnp.where(valid, sel_y2, 0.0)


def _merge_kernel(s_ref, x1_ref, y1_ref, x2_ref, y2_ref, c_ref,
                  os_ref, ox1_ref, oy1_ref, ox2_ref, oy2_ref,
                  oc_ref, on_ref):
    s = s_ref[...]
    x1 = x1_ref[...]
    y1 = y1_ref[...]
    x2 = x2_ref[...]
    y2 = y2_ref[...]
    cls = c_ref[...]
    iota = jax.lax.broadcasted_iota(jnp.int32, s.shape, 1)
    big = jnp.int32(1 << 30)
    nval = jnp.zeros((s.shape[0], 1), jnp.float32)

    for t in range(_MAX_TOTAL):
        m = jnp.max(s, axis=1, keepdims=True)
        valid = m > _NEG_INF
        idxv = jnp.min(jnp.where(s == m, iota, big), axis=1, keepdims=True)
        oh = iota == idxv
        sel_x1 = jnp.sum(jnp.where(oh, x1, 0.0), axis=1, keepdims=True)
        sel_y1 = jnp.sum(jnp.where(oh, y1, 0.0), axis=1, keepdims=True)
        sel_x2 = jnp.sum(jnp.where(oh, x2, 0.0), axis=1, keepdims=True)
        sel_y2 = jnp.sum(jnp.where(oh, y2, 0.0), axis=1, keepdims=True)
        sel_c = jnp.sum(jnp.where(oh, cls, 0.0), axis=1, keepdims=True)

        os_ref[:, t:t + 1] = jnp.where(valid, m, 0.0)
        ox1_ref[:, t:t + 1] = jnp.where(valid, sel_x1, 0.0)
        oy1_ref[:, t:t + 1] = jnp.where(valid, sel_y1, 0.0)
        ox2_ref[:, t:t + 1] = jnp.where(valid, sel_x2, 0.0)
        oy2_ref[:, t:t + 1] = jnp.where(valid, sel_y2, 0.0)
        oc_ref[:, t:t + 1] = jnp.where(valid, sel_c, 0.0)
        nval = nval + jnp.where(valid, 1.0, 0.0)
        s = jnp.where(oh, _NEG_INF, s)

    on_ref[...] = nval


def kernel(predictions, anchors):
    B, N, F = predictions.shape
    C = F - 4
    P = B * C

    logits_t = jnp.transpose(predictions[..., 4:], (0, 2, 1)).reshape(P, N)
    tile = 64
    masked = pl.pallas_call(
        _score_mask_kernel,
        grid=(P // tile,),
        in_specs=[pl.BlockSpec((tile, N), lambda i: (i, 0))],
        out_specs=pl.BlockSpec((tile, N), lambda i: (i, 0)),
        out_shape=jax.ShapeDtypeStruct((P, N), jnp.float32),
    )(logits_t)

    top_s, top_i = jax.lax.top_k(masked, _K_PRE)
    idx = top_i.reshape(B, C, _K_PRE)
    pred4 = predictions[..., :4]
    g = jnp.take_along_axis(pred4[:, None], idx[..., None], axis=2)
    ga = jnp.take(anchors, idx, axis=0)
    px = g[..., 0].reshape(P, _K_PRE)
    py = g[..., 1].reshape(P, _K_PRE)
    pw = g[..., 2].reshape(P, _K_PRE)
    ph = g[..., 3].reshape(P, _K_PRE)
    ax = ga[..., 0].reshape(P, _K_PRE)
    ay = ga[..., 1].reshape(P, _K_PRE)
    aw = ga[..., 2].reshape(P, _K_PRE)
    ah = ga[..., 3].reshape(P, _K_PRE)

    pt = 16
    row_spec = pl.BlockSpec((pt, _K_PRE), lambda i: (i, 0))
    out_row_spec = pl.BlockSpec((pt, _MAX_PER_CLASS), lambda i: (i, 0))
    ks, kx1, ky1, kx2, ky2 = pl.pallas_call(
        _nms_kernel,
        grid=(P // pt,),
        in_specs=[row_spec] * 9,
        out_specs=[out_row_spec] * 5,
        out_shape=[jax.ShapeDtypeStruct((P, _MAX_PER_CLASS), jnp.float32)] * 5,
    )(top_s, px, py, pw, ph, ax, ay, aw, ah)

    M = C * _MAX_PER_CLASS
    fs = ks.reshape(B, M)
    fx1 = kx1.reshape(B, M)
    fy1 = ky1.reshape(B, M)
    fx2 = kx2.reshape(B, M)
    fy2 = ky2.reshape(B, M)
    fcls = jnp.broadcast_to(
        jnp.repeat(jnp.arange(C, dtype=jnp.float32), _MAX_PER_CLASS)[None, :],
        (B, M))

    full = pl.BlockSpec((B, M), lambda: (0, 0))
    out_full = pl.BlockSpec((B, _MAX_TOTAL), lambda: (0, 0))
    os_, ox1, oy1, ox2, oy2, ocls, onv = pl.pallas_call(
        _merge_kernel,
        grid=(),
        in_specs=[full] * 6,
        out_specs=[out_full] * 6 + [pl.BlockSpec((B, 1), lambda: (0, 0))],
        out_shape=[jax.ShapeDtypeStruct((B, _MAX_TOTAL), jnp.float32)] * 6
        + [jax.ShapeDtypeStruct((B, 1), jnp.float32)],
    )(fs, fx1, fy1, fx2, fy2, fcls)

    out_boxes = jnp.stack([ox1, oy1, ox2, oy2], axis=-1)
    n_valid = onv.reshape(B).astype(jnp.int32)
    return out_boxes, os_, ocls, n_valid


# speedup vs baseline: 2.0334x; 1.1206x over previous
import jax
import jax.numpy as jnp
from jax.experimental import pallas as pl

_NMS_IOU = 0.1
_MAX_PER_CLASS = 50
_MAX_TOTAL = 50
_K_PRE = 1000
_NEG_INF = float("-inf")


def _score_mask_kernel(logit_ref, out_ref):
    p = jax.nn.sigmoid(logit_ref[...])
    out_ref[...] = jnp.where(p >= 0.5, p, _NEG_INF)


def _nms_kernel(s_ref, px_ref, py_ref, pw_ref, ph_ref,
                ax_ref, ay_ref, aw_ref, ah_ref,
                os_ref, ox1_ref, oy1_ref, ox2_ref, oy2_ref):
    s = s_ref[...]
    cx = px_ref[...] * aw_ref[...] + ax_ref[...]
    cy = py_ref[...] * ah_ref[...] + ay_ref[...]
    w = jnp.exp(pw_ref[...]) * aw_ref[...]
    h = jnp.exp(ph_ref[...]) * ah_ref[...]
    x1 = cx - w / 2.0
    y1 = cy - h / 2.0
    x2 = cx + w / 2.0
    y2 = cy + h / 2.0
    area = jnp.maximum(x2 - x1, 0.0) * jnp.maximum(y2 - y1, 0.0)

    iota = jax.lax.broadcasted_iota(jnp.int32, s.shape, 1)
    big = jnp.int32(1 << 30)
    alive = jnp.ones(s.shape, jnp.bool_)

    for t in range(_MAX_PER_CLASS):
        idxv = jnp.min(jnp.where(alive, iota, big), axis=1, keepdims=True)
        oh = iota == idxv
        sel_s = jnp.max(jnp.where(oh, s, _NEG_INF), axis=1, keepdims=True)
        valid = sel_s > _NEG_INF
        sel_x1 = jnp.sum(jnp.where(oh, x1, 0.0), axis=1, keepdims=True)
        sel_y1 = jnp.sum(jnp.where(oh, y1, 0.0), axis=1, keepdims=True)
        sel_x2 = jnp.sum(jnp.where(oh, x2, 0.0), axis=1, keepdims=True)
        sel_y2 = jnp.sum(jnp.where(oh, y2, 0.0), axis=1, keepdims=True)
        sel_a = jnp.sum(jnp.where(oh, area, 0.0), axis=1, keepdims=True)

        ix1 = jnp.maximum(sel_x1, x1)
        iy1 = jnp.maximum(sel_y1, y1)
        ix2 = jnp.minimum(sel_x2, x2)
        iy2 = jnp.minimum(sel_y2, y2)
        inter = jnp.maximum(ix2 - ix1, 0.0) * jnp.maximum(iy2 - iy1, 0.0)
        iou = inter / (sel_a + area - inter + 1e-8)
        alive = alive & ~((iou > _NMS_IOU) & valid) & ~oh

        os_ref[:, t:t + 1] = jnp.where(valid, sel_s, _NEG_INF)
        ox1_ref[:, t:t + 1] = jnp.where(valid, sel_x1, 0.0)
        oy1_ref[:, t:t + 1] = jnp.where(valid, sel_y1, 0.0)
        ox2_ref[:, t:t + 1] = jnp.where(valid, sel_x2, 0.0)
        oy2_ref[:, t:t + 1] = jnp.where(valid, sel_y2, 0.0)


def _merge_kernel(s_ref, x1_ref, y1_ref, x2_ref, y2_ref, c_ref,
                  os_ref, ox1_ref, oy1_ref, ox2_ref, oy2_ref,
                  oc_ref, on_ref):
    s = s_ref[...]
    x1 = x1_ref[...]
    y1 = y1_ref[...]
    x2 = x2_ref[...]
    y2 = y2_ref[...]
    cls = c_ref[...]
    iota = jax.lax.broadcasted_iota(jnp.int32, s.shape, 1)
    big = jnp.int32(1 << 30)
    nval = jnp.zeros((s.shape[0], 1), jnp.float32)

    for t in range(_MAX_TOTAL):
        m = jnp.max(s, axis=1, keepdims=True)
        valid = m > _NEG_INF
        idxv = jnp.min(jnp.where(s == m, iota, big), axis=1, keepdims=True)
        oh = iota == idxv
        sel_x1 = jnp.sum(jnp.where(oh, x1, 0.0), axis=1, keepdims=True)
        sel_y1 = jnp.sum(jnp.where(oh, y1, 0.0), axis=1, keepdims=True)
        sel_x2 = jnp.sum(jnp.where(oh, x2, 0.0), axis=1, keepdims=True)
        sel_y2 = jnp.sum(jnp.where(oh, y2, 0.0), axis=1, keepdims=True)
        sel_c = jnp.sum(jnp.where(oh, cls, 0.0), axis=1, keepdims=True)

        os_ref[:, t:t + 1] = jnp.where(valid, m, 0.0)
        ox1_ref[:, t:t + 1] = jnp.where(valid, sel_x1, 0.0)
        oy1_ref[:, t:t + 1] = jnp.where(valid, sel_y1, 0.0)
        ox2_ref[:, t:t + 1] = jnp.where(valid, sel_x2, 0.0)
        oy2_ref[:, t:t + 1] = jnp.where(valid, sel_y2, 0.0)
        oc_ref[:, t:t + 1] = jnp.where(valid, sel_c, 0.0)
        nval = nval + jnp.where(valid, 1.0, 0.0)
        s = jnp.where(oh, _NEG_INF, s)

    on_ref[...] = nval


def kernel(predictions, anchors):
    B, N, F = predictions.shape
    C = F - 4
    P = B * C

    logits_t = jnp.transpose(predictions[..., 4:], (0, 2, 1)).reshape(P, N)
    tile = 64
    masked = pl.pallas_call(
        _score_mask_kernel,
        grid=(P // tile,),
        in_specs=[pl.BlockSpec((tile, N), lambda i: (i, 0))],
        out_specs=pl.BlockSpec((tile, N), lambda i: (i, 0)),
        out_shape=jax.ShapeDtypeStruct((P, N), jnp.float32),
    )(logits_t)

    top_s, top_i = jax.lax.top_k(masked, _K_PRE)
    idx = top_i.reshape(B, C, _K_PRE)
    pred4 = predictions[..., :4]
    table = jnp.concatenate(
        [pred4, jnp.broadcast_to(anchors[None], (B, N, 4))], axis=-1
    ).reshape(B * N, 8)
    gidx = (idx + (jnp.arange(B, dtype=jnp.int32) * N)[:, None, None]).reshape(-1)
    g8 = jnp.take(table, gidx, axis=0).reshape(P, _K_PRE, 8)
    px = g8[..., 0]
    py = g8[..., 1]
    pw = g8[..., 2]
    ph = g8[..., 3]
    ax = g8[..., 4]
    ay = g8[..., 5]
    aw = g8[..., 6]
    ah = g8[..., 7]

    pt = 16
    row_spec = pl.BlockSpec((pt, _K_PRE), lambda i: (i, 0))
    out_row_spec = pl.BlockSpec((pt, _MAX_PER_CLASS), lambda i: (i, 0))
    ks, kx1, ky1, kx2, ky2 = pl.pallas_call(
        _nms_kernel,
        grid=(P // pt,),
        in_specs=[row_spec] * 9,
        out_specs=[out_row_spec] * 5,
        out_shape=[jax.ShapeDtypeStruct((P, _MAX_PER_CLASS), jnp.float32)] * 5,
    )(top_s, px, py, pw, ph, ax, ay, aw, ah)

    M = C * _MAX_PER_CLASS
    fs = ks.reshape(B, M)
    fx1 = kx1.reshape(B, M)
    fy1 = ky1.reshape(B, M)
    fx2 = kx2.reshape(B, M)
    fy2 = ky2.reshape(B, M)
    fcls = jnp.broadcast_to(
        jnp.repeat(jnp.arange(C, dtype=jnp.float32), _MAX_PER_CLASS)[None, :],
        (B, M))

    full = pl.BlockSpec((B, M), lambda: (0, 0))
    out_full = pl.BlockSpec((B, _MAX_TOTAL), lambda: (0, 0))
    os_, ox1, oy1, ox2, oy2, ocls, onv = pl.pallas_call(
        _merge_kernel,
        grid=(),
        in_specs=[full] * 6,
        out_specs=[out_full] * 6 + [pl.BlockSpec((B, 1), lambda: (0, 0))],
        out_shape=[jax.ShapeDtypeStruct((B, _MAX_TOTAL), jnp.float32)] * 6
        + [jax.ShapeDtypeStruct((B, 1), jnp.float32)],
    )(fs, fx1, fy1, fx2, fy2, fcls)

    out_boxes = jnp.stack([ox1, oy1, ox2, oy2], axis=-1)
    n_valid = onv.reshape(B).astype(jnp.int32)
    return out_boxes, os_, ocls, n_valid


# speedup vs baseline: 3.5996x; 1.7702x over previous
import functools

import jax
import jax.numpy as jnp
from jax import lax
from jax.experimental import pallas as pl
from jax.experimental.pallas import tpu as pltpu, tpu_sc as plsc

_NMS_IOU = 0.1
_MAX_PER_CLASS = 50
_MAX_TOTAL = 50
_K_PRE = 1000
_NEG_INF = float("-inf")


def _score_mask_kernel(logit_ref, out_ref):
    p = jax.nn.sigmoid(logit_ref[...])
    out_ref[...] = jnp.where(p >= 0.5, p, _NEG_INF)


def _nms_kernel(s_ref, px_ref, py_ref, pw_ref, ph_ref,
                ax_ref, ay_ref, aw_ref, ah_ref,
                os_ref, ox1_ref, oy1_ref, ox2_ref, oy2_ref):
    s = s_ref[...]
    cx = px_ref[...] * aw_ref[...] + ax_ref[...]
    cy = py_ref[...] * ah_ref[...] + ay_ref[...]
    w = jnp.exp(pw_ref[...]) * aw_ref[...]
    h = jnp.exp(ph_ref[...]) * ah_ref[...]
    x1 = cx - w / 2.0
    y1 = cy - h / 2.0
    x2 = cx + w / 2.0
    y2 = cy + h / 2.0
    area = jnp.maximum(x2 - x1, 0.0) * jnp.maximum(y2 - y1, 0.0)

    iota = jax.lax.broadcasted_iota(jnp.int32, s.shape, 1)
    big = jnp.int32(1 << 30)
    alive = jnp.ones(s.shape, jnp.bool_)

    for t in range(_MAX_PER_CLASS):
        idxv = jnp.min(jnp.where(alive, iota, big), axis=1, keepdims=True)
        oh = iota == idxv
        sel_s = jnp.max(jnp.where(oh, s, _NEG_INF), axis=1, keepdims=True)
        valid = sel_s > _NEG_INF
        sel_x1 = jnp.sum(jnp.where(oh, x1, 0.0), axis=1, keepdims=True)
        sel_y1 = jnp.sum(jnp.where(oh, y1, 0.0), axis=1, keepdims=True)
        sel_x2 = jnp.sum(jnp.where(oh, x2, 0.0), axis=1, keepdims=True)
        sel_y2 = jnp.sum(jnp.where(oh, y2, 0.0), axis=1, keepdims=True)
        sel_a = jnp.sum(jnp.where(oh, area, 0.0), axis=1, keepdims=True)

        ix1 = jnp.maximum(sel_x1, x1)
        iy1 = jnp.maximum(sel_y1, y1)
        ix2 = jnp.minimum(sel_x2, x2)
        iy2 = jnp.minimum(sel_y2, y2)
        inter = jnp.maximum(ix2 - ix1, 0.0) * jnp.maximum(iy2 - iy1, 0.0)
        iou = inter / (sel_a + area - inter + 1e-8)
        alive = alive & ~((iou > _NMS_IOU) & valid) & ~oh

        os_ref[:, t:t + 1] = jnp.where(valid, sel_s, _NEG_INF)
        ox1_ref[:, t:t + 1] = jnp.where(valid, sel_x1, 0.0)
        oy1_ref[:, t:t + 1] = jnp.where(valid, sel_y1, 0.0)
        ox2_ref[:, t:t + 1] = jnp.where(valid, sel_x2, 0.0)
        oy2_ref[:, t:t + 1] = jnp.where(valid, sel_y2, 0.0)


def _merge_kernel(s_ref, x1_ref, y1_ref, x2_ref, y2_ref, c_ref,
                  os_ref, ox1_ref, oy1_ref, ox2_ref, oy2_ref,
                  oc_ref, on_ref):
    s = s_ref[...]
    x1 = x1_ref[...]
    y1 = y1_ref[...]
    x2 = x2_ref[...]
    y2 = y2_ref[...]
    cls = c_ref[...]
    iota = jax.lax.broadcasted_iota(jnp.int32, s.shape, 1)
    big = jnp.int32(1 << 30)
    nval = jnp.zeros((s.shape[0], 1), jnp.float32)

    for t in range(_MAX_TOTAL):
        m = jnp.max(s, axis=1, keepdims=True)
        valid = m > _NEG_INF
        idxv = jnp.min(jnp.where(s == m, iota, big), axis=1, keepdims=True)
        oh = iota == idxv
        sel_x1 = jnp.sum(jnp.where(oh, x1, 0.0), axis=1, keepdims=True)
        sel_y1 = jnp.sum(jnp.where(oh, y1, 0.0), axis=1, keepdims=True)
        sel_x2 = jnp.sum(jnp.where(oh, x2, 0.0), axis=1, keepdims=True)
        sel_y2 = jnp.sum(jnp.where(oh, y2, 0.0), axis=1, keepdims=True)
        sel_c = jnp.sum(jnp.where(oh, cls, 0.0), axis=1, keepdims=True)

        os_ref[:, t:t + 1] = jnp.where(valid, m, 0.0)
        ox1_ref[:, t:t + 1] = jnp.where(valid, sel_x1, 0.0)
        oy1_ref[:, t:t + 1] = jnp.where(valid, sel_y1, 0.0)
        ox2_ref[:, t:t + 1] = jnp.where(valid, sel_x2, 0.0)
        oy2_ref[:, t:t + 1] = jnp.where(valid, sel_y2, 0.0)
        oc_ref[:, t:t + 1] = jnp.where(valid, sel_c, 0.0)
        nval = nval + jnp.where(valid, 1.0, 0.0)
        s = jnp.where(oh, _NEG_INF, s)

    on_ref[...] = nval


def kernel(predictions, anchors):
    B, N, F = predictions.shape
    C = F - 4
    P = B * C

    logits_t = jnp.transpose(predictions[..., 4:], (0, 2, 1)).reshape(P, N)
    tile = 64
    masked = pl.pallas_call(
        _score_mask_kernel,
        grid=(P // tile,),
        in_specs=[pl.BlockSpec((tile, N), lambda i: (i, 0))],
        out_specs=pl.BlockSpec((tile, N), lambda i: (i, 0)),
        out_shape=jax.ShapeDtypeStruct((P, N), jnp.float32),
    )(logits_t)

    top_s, top_i = jax.lax.top_k(masked, _K_PRE)
    idx = top_i.reshape(B, C, _K_PRE)
    pred4 = predictions[..., :4]
    _D = 128
    table = jnp.concatenate(
        [pred4, jnp.broadcast_to(anchors[None], (B, N, 4)),
         jnp.zeros((B, N, _D - 8), jnp.float32)], axis=-1
    ).reshape(B * N, _D)
    gidx = (idx + (jnp.arange(B, dtype=jnp.int32) * N)[:, None, None]).reshape(-1)

    nb = gidx.shape[0]
    info = plsc.get_sparse_core_info()
    nw = info.num_cores * info.num_subcores
    b_per_w = nb // nw
    chunk = 1000
    n_chunks = b_per_w // chunk
    mesh = plsc.VectorSubcoreMesh(core_axis_name="c", subcore_axis_name="s")

    @functools.partial(
        pl.kernel, mesh=mesh,
        out_type=jax.ShapeDtypeStruct((nb, _D), jnp.float32),
        scratch_types=[
            pltpu.VMEM((chunk,), jnp.int32),
            pltpu.VMEM((chunk, _D), jnp.float32),
            pltpu.SemaphoreType.DMA,
        ],
    )
    def _sc_gather(table_hbm, idx_hbm, out_hbm, idx_v, rows_v, sem):
        wid = lax.axis_index("s") * info.num_cores + lax.axis_index("c")
        base = wid * b_per_w
        for j in range(n_chunks):
            off = base + j * chunk
            pltpu.sync_copy(idx_hbm.at[pl.ds(off, chunk)], idx_v)
            pltpu.async_copy(table_hbm.at[idx_v], rows_v, sem).wait()
            pltpu.sync_copy(rows_v, out_hbm.at[pl.ds(off, chunk)])

    g8 = _sc_gather(table, gidx).reshape(P, _K_PRE, _D)
    px = g8[..., 0]
    py = g8[..., 1]
    pw = g8[..., 2]
    ph = g8[..., 3]
    ax = g8[..., 4]
    ay = g8[..., 5]
    aw = g8[..., 6]
    ah = g8[..., 7]

    pt = 16
    row_spec = pl.BlockSpec((pt, _K_PRE), lambda i: (i, 0))
    out_row_spec = pl.BlockSpec((pt, _MAX_PER_CLASS), lambda i: (i, 0))
    ks, kx1, ky1, kx2, ky2 = pl.pallas_call(
        _nms_kernel,
        grid=(P // pt,),
        in_specs=[row_spec] * 9,
        out_specs=[out_row_spec] * 5,
        out_shape=[jax.ShapeDtypeStruct((P, _MAX_PER_CLASS), jnp.float32)] * 5,
    )(top_s, px, py, pw, ph, ax, ay, aw, ah)

    M = C * _MAX_PER_CLASS
    fs = ks.reshape(B, M)
    fx1 = kx1.reshape(B, M)
    fy1 = ky1.reshape(B, M)
    fx2 = kx2.reshape(B, M)
    fy2 = ky2.reshape(B, M)
    fcls = jnp.broadcast_to(
        jnp.repeat(jnp.arange(C, dtype=jnp.float32), _MAX_PER_CLASS)[None, :],
        (B, M))

    full = pl.BlockSpec((B, M), lambda: (0, 0))
    out_full = pl.BlockSpec((B, _MAX_TOTAL), lambda: (0, 0))
    os_, ox1, oy1, ox2, oy2, ocls, onv = pl.pallas_call(
        _merge_kernel,
        grid=(),
        in_specs=[full] * 6,
        out_specs=[out_full] * 6 + [pl.BlockSpec((B, 1), lambda: (0, 0))],
        out_shape=[jax.ShapeDtypeStruct((B, _MAX_TOTAL), jnp.float32)] * 6
        + [jax.ShapeDtypeStruct((B, 1), jnp.float32)],
    )(fs, fx1, fy1, fx2, fy2, fcls)

    out_boxes = jnp.stack([ox1, oy1, ox2, oy2], axis=-1)
    n_valid = onv.reshape(B).astype(jnp.int32)
    return out_boxes, os_, ocls, n_valid
